# baseline (speedup 1.0000x reference)
import jax
import jax.numpy as jnp
from jax import lax
from jax.experimental import pallas as pl
from jax.experimental.pallas import tpu as pltpu
from jax.experimental.pallas import tpu_sc as plsc

N = 10000
D = 128
N_PAD = 10112
NC = 2
NS = 16
NW = NC * NS
C = 32
CH = 320
NB = 8
LG = 6
H = 16
CHH = CH // H
EPT = C * CH
E_PAD = NW * EPT
ROWS_PER_TILE = N_PAD // NS
DR = 80
DEG_OFF = 10016
DST_PAD = 10000


def _sc_scatter_body(feat_hbm, src_hbm, dst_hbm, out_hbm,
                     acc, srcv, dstv, deg_pr, buf, *sems):
    cid = lax.axis_index("c")
    sid = lax.axis_index("s")
    wid = cid * NS + sid
    class _SemArr:
        def __init__(self, lst):
            self.lst = lst

        def at(self, i):
            return self.lst[i]

    gsem = _SemArr(sems[:NB])
    ssem = _SemArr(sems[NB:2 * NB])

    zv = jnp.zeros((16,), jnp.float32)

    def zero_buf0(r, _):
        for k in range(D // 16):
            buf[0, r, pl.ds(k * 16, 16)] = zv
        return 0

    lax.fori_loop(0, C, zero_buf0, 0)

    def zero_deg(r, _):
        for k in range(D // 16):
            deg_pr[r, pl.ds(k * 16, 16)] = zv
        return 0

    lax.fori_loop(0, DR, zero_deg, 0)

    base = sid * ROWS_PER_TILE
    nz = ROWS_PER_TILE // C
    for k in range(nz):
        pltpu.async_copy(buf.at[0], acc.at[pl.ds(base + k * C, C)],
                         gsem.at(0))
    rem = ROWS_PER_TILE % C
    if rem:
        pltpu.sync_copy(buf.at[0, pl.ds(0, rem)],
                        acc.at[pl.ds(base + ROWS_PER_TILE - rem, rem)])
    for k in range(nz):
        pltpu.make_async_copy(buf.at[0], acc.at[pl.ds(base + k * C, C)],
                              gsem.at(0)).wait()

    plsc.subcore_barrier()

    ones16 = jnp.ones((16,), jnp.float32)

    def count_deg(m):
        for g in range(C // 16):
            d16 = dstv[m, pl.ds(g * 16, 16)]
            plsc.addupdate_scatter(
                deg_pr, [lax.shift_right_logical(d16, 7),
                         lax.bitwise_and(d16, 127)],
                ones16)

    def gather(m):
        pltpu.async_copy(feat_hbm.at[srcv.at[m]], buf.at[m % NB],
                         gsem.at(m % NB))

    def gather_wait(m):
        pltpu.make_async_copy(feat_hbm.at[srcv.at[m]], buf.at[m % NB],
                              gsem.at(m % NB)).wait()

    def scatter(m):
        pltpu.async_copy(buf.at[m % NB], acc.at[dstv.at[m]],
                         ssem.at(m % NB), add=True)

    def scatter_wait(m):
        pltpu.make_async_copy(buf.at[m % NB], acc.at[dstv.at[m]],
                              ssem.at(m % NB)).wait()


    pltpu.sync_copy(src_hbm.at[wid, 0], srcv)
    pltpu.sync_copy(dst_hbm.at[wid, 0], dstv)

    def slice_body(h, _):
        for m in range(LG):
            gather(m)
        for m in range(CHH):
            gather_wait(m)
            scatter(m)
            count_deg(m)
            if m + LG < CHH:
                if m >= NB - LG:
                    scatter_wait(m - (NB - LG))
                gather(m + LG)

        @pl.when(h + 1 < H)
        def _():
            pltpu.async_copy(src_hbm.at[wid, h + 1], srcv, gsem.at(0))
        for m in range(CHH - NB, CHH):
            scatter_wait(m)

        @pl.when(h + 1 < H)
        def _():
            pltpu.sync_copy(dst_hbm.at[wid, h + 1], dstv)
            pltpu.make_async_copy(src_hbm.at[wid, h + 1], srcv,
                                  gsem.at(0)).wait()
        return 0

    lax.fori_loop(0, H, slice_body, 0)

    iota16 = lax.iota(jnp.int32, 16)
    for k in range(DR // 16):
        pltpu.sync_copy(deg_pr.at[pl.ds(k * 16, 16)],
                        acc.at[DEG_OFF + k * 16 + iota16], add=True)

    plsc.subcore_barrier()
    pltpu.sync_copy(acc.at[pl.ds(base, ROWS_PER_TILE)],
                    out_hbm.at[cid, pl.ds(base, ROWS_PER_TILE)])


_sc_scatter = pl.kernel(
    _sc_scatter_body,
    out_type=jax.ShapeDtypeStruct((NC, N_PAD, D), jnp.float32),
    mesh=plsc.VectorSubcoreMesh(core_axis_name="c", subcore_axis_name="s"),
    compiler_params=pltpu.CompilerParams(needs_layout_passes=False),
    scratch_types=[
        pltpu.VMEM_SHARED((N_PAD, D), jnp.float32),
        pltpu.VMEM((CHH, C), jnp.int32),
        pltpu.VMEM((CHH, C), jnp.int32),
        pltpu.VMEM((DR, D), jnp.float32),
        pltpu.VMEM((NB, C, D), jnp.float32),
    ] + [pltpu.SemaphoreType.DMA] * (2 * NB),
)


BN = 1024
N_OUT = 10240


def _tc_epilogue_body(parts_ref, deg_ref, feat_ref, wsT_ref, wnT_ref,
                      bs_ref, bn_ref, out_ref):
    p = parts_ref[...]
    summed = p[0] + p[1]
    dp = deg_ref[...]
    deg = dp[0] + dp[1]
    scale = 1.0 / jnp.maximum(deg, 1.0)
    x = feat_ref[...]
    out_ref[...] = (
        jnp.dot(x, wsT_ref[...], preferred_element_type=jnp.float32)
        + bs_ref[...]
        + scale * jnp.dot(summed, wnT_ref[...], preferred_element_type=jnp.float32)
        + bn_ref[...]
    )


def _tc_epilogue(parts, deg, feat_pad, wsT, wnT, bs, bn):
    return pl.pallas_call(
        _tc_epilogue_body,
        grid=(N_OUT // BN,),
        in_specs=[
            pl.BlockSpec((NC, BN, D), lambda i: (0, i, 0)),
            pl.BlockSpec((NC, BN, 1), lambda i: (0, i, 0)),
            pl.BlockSpec((BN, D), lambda i: (i, 0)),
            pl.BlockSpec((D, D), lambda i: (0, 0)),
            pl.BlockSpec((D, D), lambda i: (0, 0)),
            pl.BlockSpec((1, D), lambda i: (0, 0)),
            pl.BlockSpec((1, D), lambda i: (0, 0)),
        ],
        out_specs=pl.BlockSpec((BN, D), lambda i: (i, 0)),
        out_shape=jax.ShapeDtypeStruct((N_OUT, D), jnp.float32),
    )(parts, deg, feat_pad, wsT, wnT, bs, bn)


def kernel(feat, edge_index, node_type, W_self, b_self, W_neigh, b_neigh):
    del node_type
    E = edge_index.shape[1]
    src = edge_index[0].astype(jnp.int32)
    dst = edge_index[1].astype(jnp.int32)
    ppt = EPT - E // NW
    src_p = jnp.concatenate(
        [src.reshape(NW, E // NW), jnp.zeros((NW, ppt), jnp.int32)], axis=1)
    padd = DST_PAD + (jnp.arange(ppt, dtype=jnp.int32) % 16)
    dst_p = jnp.concatenate(
        [dst.reshape(NW, E // NW), jnp.broadcast_to(padd, (NW, ppt))], axis=1)
    src3 = src_p.reshape(NW, H, CHH, C)
    dst3 = dst_p.reshape(NW, H, CHH, C)

    parts = _sc_scatter(feat, src3, dst3)

    deg = parts[:, DEG_OFF:DEG_OFF + DR, :].reshape(NC, DR * D, 1)
    feat_pad = jnp.pad(feat, ((0, N_OUT - N), (0, 0)))
    out = _tc_epilogue(parts, deg, feat_pad, W_self.T, W_neigh.T,
                       b_self[None, :], b_neigh[None, :])
    return out[:N]


# speedup vs baseline: 1.0019x; 1.0019x over previous
import jax
import jax.numpy as jnp
from jax import lax
from jax.experimental import pallas as pl
from jax.experimental.pallas import tpu as pltpu
from jax.experimental.pallas import tpu_sc as plsc

N = 10000
D = 128
N_PAD = 10112
NC = 2
NS = 16
NW = NC * NS
C = 32
CH = 320
NB = 8
LG = 6
H = 16
CHH = CH // H
EPT = C * CH
E_PAD = NW * EPT
ROWS_PER_TILE = N_PAD // NS
DR = 80
DEG_OFF = 10016
DST_PAD = 10000


def _sc_scatter_body(feat_hbm, src_hbm, dst_hbm, out_hbm,
                     acc, srcv, dstv, deg_pr, buf, *sems):
    cid = lax.axis_index("c")
    sid = lax.axis_index("s")
    wid = cid * NS + sid
    class _SemArr:
        def __init__(self, lst):
            self.lst = lst

        def at(self, i):
            return self.lst[i]

    gsem = _SemArr(sems[:NB])
    ssem = _SemArr(sems[NB:2 * NB])

    zv = jnp.zeros((16,), jnp.float32)

    def zero_buf0(r, _):
        for k in range(D // 16):
            buf[0, r, pl.ds(k * 16, 16)] = zv
        return 0

    lax.fori_loop(0, C, zero_buf0, 0)

    def zero_deg(r, _):
        for k in range(D // 16):
            deg_pr[r, pl.ds(k * 16, 16)] = zv
        return 0

    lax.fori_loop(0, DR, zero_deg, 0)

    base = sid * ROWS_PER_TILE
    nz = ROWS_PER_TILE // C
    for k in range(nz):
        pltpu.async_copy(buf.at[0], acc.at[pl.ds(base + k * C, C)],
                         gsem.at(0))
    rem = ROWS_PER_TILE % C
    if rem:
        pltpu.sync_copy(buf.at[0, pl.ds(0, rem)],
                        acc.at[pl.ds(base + ROWS_PER_TILE - rem, rem)])
    for k in range(nz):
        pltpu.make_async_copy(buf.at[0], acc.at[pl.ds(base + k * C, C)],
                              gsem.at(0)).wait()

    plsc.subcore_barrier()

    ones16 = jnp.ones((16,), jnp.float32)

    def count_deg(m):
        for g in range(C // 16):
            d16 = dstv[m, pl.ds(g * 16, 16)]
            plsc.addupdate_scatter(
                deg_pr, [lax.shift_right_logical(d16, 7),
                         lax.bitwise_and(d16, 127)],
                ones16)

    def gather(m):
        pltpu.async_copy(feat_hbm.at[srcv.at[m]], buf.at[m % NB],
                         gsem.at(m % NB))

    def gather_wait(m):
        pltpu.make_async_copy(feat_hbm.at[srcv.at[m]], buf.at[m % NB],
                              gsem.at(m % NB)).wait()

    def scatter(m):
        pltpu.async_copy(buf.at[m % NB], acc.at[dstv.at[m]],
                         ssem.at(m % NB), add=True)

    def scatter_wait(m):
        pltpu.make_async_copy(buf.at[m % NB], acc.at[dstv.at[m]],
                              ssem.at(m % NB)).wait()


    pltpu.sync_copy(src_hbm.at[wid, 0], srcv)
    pltpu.sync_copy(dst_hbm.at[wid, 0], dstv)

    def slice_body(h, _):
        for m in range(LG):
            gather(m)
        for m in range(CHH):
            gather_wait(m)
            scatter(m)
            count_deg(m)
            if m + LG < CHH:
                if m >= NB - LG:
                    scatter_wait(m - (NB - LG))
                gather(m + LG)

        @pl.when(h + 1 < H)
        def _():
            pltpu.async_copy(src_hbm.at[wid, h + 1], srcv, gsem.at(0))
        for m in range(CHH - NB, CHH):
            scatter_wait(m)

        @pl.when(h + 1 < H)
        def _():
            pltpu.sync_copy(dst_hbm.at[wid, h + 1], dstv)
            pltpu.make_async_copy(src_hbm.at[wid, h + 1], srcv,
                                  gsem.at(0)).wait()
        return 0

    lax.fori_loop(0, H, slice_body, 0)

    iota16 = lax.iota(jnp.int32, 16)
    for k in range(DR // 16):
        pltpu.sync_copy(deg_pr.at[pl.ds(k * 16, 16)],
                        acc.at[DEG_OFF + k * 16 + iota16], add=True)

    plsc.subcore_barrier()
    pltpu.sync_copy(acc.at[pl.ds(base, ROWS_PER_TILE)],
                    out_hbm.at[cid, pl.ds(base, ROWS_PER_TILE)])


_sc_scatter = pl.kernel(
    _sc_scatter_body,
    out_type=jax.ShapeDtypeStruct((NC, N_PAD, D), jnp.float32),
    mesh=plsc.VectorSubcoreMesh(core_axis_name="c", subcore_axis_name="s"),
    compiler_params=pltpu.CompilerParams(needs_layout_passes=False),
    scratch_types=[
        pltpu.VMEM_SHARED((N_PAD, D), jnp.float32),
        pltpu.VMEM((CHH, C), jnp.int32),
        pltpu.VMEM((CHH, C), jnp.int32),
        pltpu.VMEM((DR, D), jnp.float32),
        pltpu.VMEM((NB, C, D), jnp.float32),
    ] + [pltpu.SemaphoreType.DMA] * (2 * NB),
)


BN = 1024
N_OUT = 10240


def _tc_epilogue_body(parts_ref, deg_ref, feat_ref, wsT_ref, wnT_ref,
                      bs_ref, bn_ref, out_ref):
    p = parts_ref[...]
    summed = p[0] + p[1]
    dp = deg_ref[...]
    deg = dp[0] + dp[1]
    scale = 1.0 / jnp.maximum(deg, 1.0)
    x = feat_ref[...]
    out_ref[...] = (
        jnp.dot(x, wsT_ref[...], preferred_element_type=jnp.float32)
        + bs_ref[...]
        + scale * jnp.dot(summed, wnT_ref[...], preferred_element_type=jnp.float32)
        + bn_ref[...]
    )


def _tc_epilogue(parts, deg, feat_pad, wsT, wnT, bs, bn):
    return pl.pallas_call(
        _tc_epilogue_body,
        grid=(N_OUT // BN,),
        in_specs=[
            pl.BlockSpec((NC, BN, D), lambda i: (0, i, 0)),
            pl.BlockSpec((NC, BN, 1), lambda i: (0, i, 0)),
            pl.BlockSpec((BN, D), lambda i: (i, 0)),
            pl.BlockSpec((D, D), lambda i: (0, 0)),
            pl.BlockSpec((D, D), lambda i: (0, 0)),
            pl.BlockSpec((1, D), lambda i: (0, 0)),
            pl.BlockSpec((1, D), lambda i: (0, 0)),
        ],
        out_specs=pl.BlockSpec((BN, D), lambda i: (i, 0)),
        out_shape=jax.ShapeDtypeStruct((N_OUT, D), jnp.float32),
    )(parts, deg, feat_pad, wsT, wnT, bs, bn)


def kernel(feat, edge_index, node_type, W_self, b_self, W_neigh, b_neigh):
    del node_type
    E = edge_index.shape[1]
    src = edge_index[0].astype(jnp.int32)
    dst = edge_index[1].astype(jnp.int32)
    ppt = EPT - E // NW
    src_p = jnp.concatenate(
        [src.reshape(NW, E // NW), jnp.zeros((NW, ppt), jnp.int32)], axis=1)
    padd = DST_PAD + (jnp.arange(ppt, dtype=jnp.int32) % 16)
    dst_p = jnp.concatenate(
        [dst.reshape(NW, E // NW), jnp.broadcast_to(padd, (NW, ppt))], axis=1)
    src3 = src_p.reshape(NW, H, CHH, C)
    dst3 = dst_p.reshape(NW, H, CHH, C)

    parts = _sc_scatter(feat, src3, dst3)

    deg = parts[:, DEG_OFF:DEG_OFF + DR, :].reshape(NC, DR * D, 1)
    out = _tc_epilogue(parts, deg, feat, W_self.T, W_neigh.T,
                       b_self[None, :], b_neigh[None, :])
    return out[:N]


# speedup vs baseline: 1.0022x; 1.0004x over previous
import jax
import jax.numpy as jnp
from jax import lax
from jax.experimental import pallas as pl
from jax.experimental.pallas import tpu as pltpu
from jax.experimental.pallas import tpu_sc as plsc

N = 10000
D = 128
N_PAD = 10112
NC = 2
NS = 16
NW = NC * NS
C = 32
CH = 320
NB = 8
LG = 6
H = 16
CHH = CH // H
EPT = C * CH
E_PAD = NW * EPT
ROWS_PER_TILE = N_PAD // NS
DR = 80
DEG_OFF = 10016
DST_PAD = 10000


def _sc_scatter_body(feat_hbm, src_hbm, dst_hbm, out_hbm,
                     acc, srcv, dstv, deg_pr, buf, *sems):
    cid = lax.axis_index("c")
    sid = lax.axis_index("s")
    wid = cid * NS + sid
    class _SemArr:
        def __init__(self, lst):
            self.lst = lst

        def at(self, i):
            return self.lst[i]

    gsem = _SemArr(sems[:NB])
    ssem = _SemArr(sems[NB:2 * NB])

    zv = jnp.zeros((16,), jnp.float32)

    def zero_buf0(r, _):
        for k in range(D // 16):
            buf[0, r, pl.ds(k * 16, 16)] = zv
        return 0

    lax.fori_loop(0, C, zero_buf0, 0)

    def zero_deg(r, _):
        for k in range(D // 16):
            deg_pr[r, pl.ds(k * 16, 16)] = zv
        return 0

    lax.fori_loop(0, DR, zero_deg, 0)

    base = sid * ROWS_PER_TILE
    nz = ROWS_PER_TILE // C
    for k in range(nz):
        pltpu.async_copy(buf.at[0], acc.at[pl.ds(base + k * C, C)],
                         gsem.at(0))
    rem = ROWS_PER_TILE % C
    if rem:
        pltpu.sync_copy(buf.at[0, pl.ds(0, rem)],
                        acc.at[pl.ds(base + ROWS_PER_TILE - rem, rem)])
    for k in range(nz):
        pltpu.make_async_copy(buf.at[0], acc.at[pl.ds(base + k * C, C)],
                              gsem.at(0)).wait()

    plsc.subcore_barrier()

    ones16 = jnp.ones((16,), jnp.float32)

    def count_deg(m):
        for g in range(C // 16):
            d16 = dstv[m, pl.ds(g * 16, 16)]
            plsc.addupdate_scatter(
                deg_pr, [lax.shift_right_logical(d16, 7),
                         lax.bitwise_and(d16, 127)],
                ones16)

    def gather(m):
        pltpu.async_copy(feat_hbm.at[srcv.at[m]], buf.at[m % NB],
                         gsem.at(m % NB))

    def gather_wait(m):
        pltpu.make_async_copy(feat_hbm.at[srcv.at[m]], buf.at[m % NB],
                              gsem.at(m % NB)).wait()

    def scatter(m):
        pltpu.async_copy(buf.at[m % NB], acc.at[dstv.at[m]],
                         ssem.at(m % NB), add=True)

    def scatter_wait(m):
        pltpu.make_async_copy(buf.at[m % NB], acc.at[dstv.at[m]],
                              ssem.at(m % NB)).wait()


    def slice_body(h, _):
        pltpu.async_copy(src_hbm.at[wid, h], srcv, gsem.at(0))
        pltpu.async_copy(dst_hbm.at[wid, h], dstv, gsem.at(1))
        pltpu.make_async_copy(src_hbm.at[wid, h], srcv, gsem.at(0)).wait()
        pltpu.make_async_copy(dst_hbm.at[wid, h], dstv, gsem.at(1)).wait()
        for m in range(LG):
            gather(m)
        for m in range(CHH):
            gather_wait(m)
            scatter(m)
            count_deg(m)
            if m + LG < CHH:
                if m >= NB - LG:
                    scatter_wait(m - (NB - LG))
                gather(m + LG)
        for m in range(CHH - NB, CHH):
            scatter_wait(m)
        return 0

    lax.fori_loop(0, H, slice_body, 0)

    iota16 = lax.iota(jnp.int32, 16)
    for k in range(DR // 16):
        pltpu.sync_copy(deg_pr.at[pl.ds(k * 16, 16)],
                        acc.at[DEG_OFF + k * 16 + iota16], add=True)

    plsc.subcore_barrier()
    pltpu.sync_copy(acc.at[pl.ds(base, ROWS_PER_TILE)],
                    out_hbm.at[cid, pl.ds(base, ROWS_PER_TILE)])


_sc_scatter = pl.kernel(
    _sc_scatter_body,
    out_type=jax.ShapeDtypeStruct((NC, N_PAD, D), jnp.float32),
    mesh=plsc.VectorSubcoreMesh(core_axis_name="c", subcore_axis_name="s"),
    compiler_params=pltpu.CompilerParams(needs_layout_passes=False),
    scratch_types=[
        pltpu.VMEM_SHARED((N_PAD, D), jnp.float32),
        pltpu.VMEM((CHH, C), jnp.int32),
        pltpu.VMEM((CHH, C), jnp.int32),
        pltpu.VMEM((DR, D), jnp.float32),
        pltpu.VMEM((NB, C, D), jnp.float32),
    ] + [pltpu.SemaphoreType.DMA] * (2 * NB),
)


BN = 1024
N_OUT = 10240


def _tc_epilogue_body(parts_ref, deg_ref, feat_ref, wsT_ref, wnT_ref,
                      bs_ref, bn_ref, out_ref):
    p = parts_ref[...]
    summed = p[0] + p[1]
    dp = deg_ref[...]
    deg = dp[0] + dp[1]
    scale = 1.0 / jnp.maximum(deg, 1.0)
    x = feat_ref[...]
    out_ref[...] = (
        jnp.dot(x, wsT_ref[...], preferred_element_type=jnp.float32)
        + bs_ref[...]
        + scale * jnp.dot(summed, wnT_ref[...], preferred_element_type=jnp.float32)
        + bn_ref[...]
    )


def _tc_epilogue(parts, deg, feat_pad, wsT, wnT, bs, bn):
    return pl.pallas_call(
        _tc_epilogue_body,
        grid=(N_OUT // BN,),
        in_specs=[
            pl.BlockSpec((NC, BN, D), lambda i: (0, i, 0)),
            pl.BlockSpec((NC, BN, 1), lambda i: (0, i, 0)),
            pl.BlockSpec((BN, D), lambda i: (i, 0)),
            pl.BlockSpec((D, D), lambda i: (0, 0)),
            pl.BlockSpec((D, D), lambda i: (0, 0)),
            pl.BlockSpec((1, D), lambda i: (0, 0)),
            pl.BlockSpec((1, D), lambda i: (0, 0)),
        ],
        out_specs=pl.BlockSpec((BN, D), lambda i: (i, 0)),
        out_shape=jax.ShapeDtypeStruct((N_OUT, D), jnp.float32),
    )(parts, deg, feat_pad, wsT, wnT, bs, bn)


def kernel(feat, edge_index, node_type, W_self, b_self, W_neigh, b_neigh):
    del node_type
    E = edge_index.shape[1]
    src = edge_index[0].astype(jnp.int32)
    dst = edge_index[1].astype(jnp.int32)
    ppt = EPT - E // NW
    src_p = jnp.concatenate(
        [src.reshape(NW, E // NW), jnp.zeros((NW, ppt), jnp.int32)], axis=1)
    padd = DST_PAD + (jnp.arange(ppt, dtype=jnp.int32) % 16)
    dst_p = jnp.concatenate(
        [dst.reshape(NW, E // NW), jnp.broadcast_to(padd, (NW, ppt))], axis=1)
    src3 = src_p.reshape(NW, H, CHH, C)
    dst3 = dst_p.reshape(NW, H, CHH, C)

    parts = _sc_scatter(feat, src3, dst3)

    deg = parts[:, DEG_OFF:DEG_OFF + DR, :].reshape(NC, DR * D, 1)
    out = _tc_epilogue(parts, deg, feat, W_self.T, W_neigh.T,
                       b_self[None, :], b_neigh[None, :])
    return out[:N]
